# baseline (speedup 1.0000x reference)
import functools

import jax
import jax.numpy as jnp
from jax import lax
from jax.experimental import pallas as pl
from jax.experimental.pallas import tpu as pltpu
from jax.experimental.pallas import tpu_sc as plsc

N = 10000
E = 320000
F = 128
NPAD = 10240
NC = 2
NS = 16
CH = 80
PER_TILE_A = E // (NC * NS)
PER_TILE_C = E // (NC * NS)
ZROWS = NPAD // NS

GW = 128
NCHUNK = 5
CL = E // NCHUNK


@functools.cache
def _make_sc_gather():
    mesh = plsc.VectorSubcoreMesh(core_axis_name="c", subcore_axis_name="s")

    @functools.partial(
        pl.kernel,
        out_type=jax.ShapeDtypeStruct((CL, F), jnp.float32),
        mesh=mesh,
    )
    def _sc_gather(x_hbm, src_hbm, out_hbm):
        def body(i_vmem, o_vmem):
            pltpu.sync_copy(x_hbm.at[i_vmem.at[0]], o_vmem)

        pltpu.emit_pipeline(
            body,
            grid=(CL // GW,),
            in_specs=[pl.BlockSpec((1, GW), lambda i: (0, i))],
            out_specs=[pl.BlockSpec((GW, F), lambda i: (i, 0))],
            core_axis_name=("c", "s"),
            dimension_semantics=(pltpu.PARALLEL,),
        )(src_hbm, out_hbm)

    return _sc_gather


EB = 1280


def _gemm_body(xg_ref, ea_ref, w_ref, msg_ref):
    f = jnp.clip(ea_ref[...], 0.0, 1.0)
    w0 = 0.5 * (1.0 - f) ** 2
    w1 = -f * f + f + 0.5
    w2 = 0.5 * f * f
    wb = (w0, w1, w2)
    xg = xg_ref[...].astype(jnp.bfloat16)
    y = jnp.concatenate(
        [
            (wb[j][:, 1:2] * wb[l][:, 2:3]).astype(jnp.bfloat16) * xg
            for j in range(3)
            for l in range(3)
        ],
        axis=1,
    )
    g = lax.dot_general(
        y, w_ref[...], (((1,), (0,)), ((), ())),
        preferred_element_type=jnp.float32,
    )
    msg_ref[...] = (
        w0[:, 0:1] * g[:, 0:128]
        + w1[:, 0:1] * g[:, 128:256]
        + w2[:, 0:1] * g[:, 256:384]
    )


def _tc_gemm(xg, ea4, wall):
    return pl.pallas_call(
        _gemm_body,
        grid=(CL // EB,),
        in_specs=[
            pl.BlockSpec((EB, F), lambda i: (i, 0)),
            pl.BlockSpec((EB, 4), lambda i: (i, 0)),
            pl.BlockSpec((9 * F, 3 * F), lambda i: (0, 0)),
        ],
        out_specs=pl.BlockSpec((EB, F), lambda i: (i, 0)),
        out_shape=jax.ShapeDtypeStruct((CL, F), jnp.float32),
    )(xg, ea4, wall)


@functools.cache
def _make_sc_scatter():
    mesh = plsc.VectorSubcoreMesh(core_axis_name="c", subcore_axis_name="s")

    @functools.partial(
        pl.kernel,
        out_type=jax.ShapeDtypeStruct((NC, NPAD, F), jnp.float32),
        mesh=mesh,
        scratch_types=[
            pltpu.VMEM_SHARED((NPAD, F), jnp.float32),
        ],
    )
    def _sc_scatter(msg_hbm, dst_hbm, zero_hbm, out_hbm, acc_sh):
        cid = lax.axis_index("c")
        sid = lax.axis_index("s")
        pltpu.sync_copy(zero_hbm, acc_sh.at[pl.ds(sid * ZROWS, ZROWS)])
        plsc.subcore_barrier()

        def body(i_vmem, m_vmem):
            pltpu.sync_copy(m_vmem, acc_sh.at[i_vmem.at[0]], add=True)

        pltpu.emit_pipeline(
            body,
            grid=(CL // GW,),
            in_specs=[
                pl.BlockSpec((1, GW), lambda i: (0, i)),
                pl.BlockSpec((GW, F), lambda i: (i, 0)),
            ],
            out_specs=[],
            core_axis_name=("c", "s"),
            dimension_semantics=(pltpu.PARALLEL,),
        )(dst_hbm, msg_hbm)

        plsc.subcore_barrier()
        pltpu.sync_copy(
            acc_sh.at[pl.ds(sid * ZROWS, ZROWS)],
            out_hbm.at[cid, pl.ds(sid * ZROWS, ZROWS)],
        )

    return _sc_scatter


def _add_body(*refs):
    o_ref = refs[-1]
    acc = jnp.sum(refs[0][...], axis=0)
    for r in refs[1:-1]:
        acc = acc + jnp.sum(r[...], axis=0)
    o_ref[...] = acc


def _tc_add(parts):
    return pl.pallas_call(
        _add_body,
        grid=(NPAD // ZROWS,),
        in_specs=[
            pl.BlockSpec((NC, ZROWS, F), lambda i: (0, i, 0)) for _ in parts
        ],
        out_specs=pl.BlockSpec((ZROWS, F), lambda i: (i, 0)),
        out_shape=jax.ShapeDtypeStruct((NPAD, F), jnp.float32),
    )(*parts)


def kernel(x, edge_index, edge_attr, W):
    src = edge_index[0].reshape(1, E)
    dst = edge_index[1].reshape(1, E)
    ea4 = jnp.pad(edge_attr, ((0, 0), (0, 1)))
    wall = (
        W.reshape(3, 3, 3, F, F)
        .transpose(1, 2, 3, 0, 4)
        .reshape(9 * F, 3 * F)
        .astype(jnp.bfloat16)
    )
    zero = jnp.zeros((ZROWS, F), jnp.float32)

    gather_k = _make_sc_gather()
    scatter_k = _make_sc_scatter()
    xgs = [
        gather_k(x, src[:, q * CL:(q + 1) * CL]) for q in range(NCHUNK)
    ]
    msgs = [
        _tc_gemm(xgs[q], ea4[q * CL:(q + 1) * CL], wall)
        for q in range(NCHUNK)
    ]
    parts = [
        scatter_k(msgs[q], dst[:, q * CL:(q + 1) * CL], zero)
        for q in range(NCHUNK)
    ]
    out = _tc_add(parts)
    return out[:N]


# speedup vs baseline: 1.0398x; 1.0398x over previous
import functools

import jax
import jax.numpy as jnp
from jax import lax
from jax.experimental import pallas as pl
from jax.experimental.pallas import tpu as pltpu
from jax.experimental.pallas import tpu_sc as plsc

N = 10000
E = 320000
F = 128
NPAD = 10240
NC = 2
NS = 16
CH = 80
PER_TILE_A = E // (NC * NS)
PER_TILE_C = E // (NC * NS)
ZROWS = NPAD // NS

GW = 128
NCHUNK = 5
CL = E // NCHUNK


@functools.cache
def _make_sc_gather():
    mesh = plsc.VectorSubcoreMesh(core_axis_name="c", subcore_axis_name="s")

    @functools.partial(
        pl.kernel,
        out_type=jax.ShapeDtypeStruct((CL, F), jnp.float32),
        mesh=mesh,
    )
    def _sc_gather(x_hbm, src_hbm, out_hbm):
        def body(i_vmem, o_vmem):
            pltpu.sync_copy(x_hbm.at[i_vmem.at[0]], o_vmem)

        pltpu.emit_pipeline(
            body,
            grid=(CL // GW,),
            in_specs=[pl.BlockSpec((1, GW), lambda i: (0, i))],
            out_specs=[pl.BlockSpec((GW, F), lambda i: (i, 0))],
            core_axis_name=("c", "s"),
            dimension_semantics=(pltpu.PARALLEL,),
        )(src_hbm, out_hbm)

    return _sc_gather


EB = 1280


NH = 2
HB = EB // NH


def _gemm_body(xg_ref, ea_ref, w_ref, msg_ref):
    for h in range(NH):
        rows = pl.ds(h * HB, HB)
        f = jnp.clip(ea_ref[rows, :], 0.0, 1.0)
        w0 = 0.5 * (1.0 - f) ** 2
        w1 = -f * f + f + 0.5
        w2 = 0.5 * f * f
        wb = (w0, w1, w2)
        xg = xg_ref[rows, :].astype(jnp.bfloat16)
        y = jnp.concatenate(
            [
                (wb[j][:, 1:2] * wb[l][:, 2:3]).astype(jnp.bfloat16) * xg
                for j in range(3)
                for l in range(3)
            ],
            axis=1,
        )
        g = lax.dot_general(
            y, w_ref[...], (((1,), (0,)), ((), ())),
            preferred_element_type=jnp.float32,
        )
        msg_ref[rows, :] = (
            w0[:, 0:1] * g[:, 0:128]
            + w1[:, 0:1] * g[:, 128:256]
            + w2[:, 0:1] * g[:, 256:384]
        )


def _tc_gemm(xg, ea4, wall):
    return pl.pallas_call(
        _gemm_body,
        grid=(CL // EB,),
        in_specs=[
            pl.BlockSpec((EB, F), lambda i: (i, 0)),
            pl.BlockSpec((EB, 4), lambda i: (i, 0)),
            pl.BlockSpec((9 * F, 3 * F), lambda i: (0, 0)),
        ],
        out_specs=pl.BlockSpec((EB, F), lambda i: (i, 0)),
        out_shape=jax.ShapeDtypeStruct((CL, F), jnp.float32),
    )(xg, ea4, wall)


@functools.cache
def _make_sc_scatter():
    mesh = plsc.VectorSubcoreMesh(core_axis_name="c", subcore_axis_name="s")

    @functools.partial(
        pl.kernel,
        out_type=jax.ShapeDtypeStruct((NC, NPAD, F), jnp.float32),
        mesh=mesh,
        scratch_types=[
            pltpu.VMEM_SHARED((NPAD, F), jnp.float32),
        ],
    )
    def _sc_scatter(msg_hbm, dst_hbm, zero_hbm, out_hbm, acc_sh):
        cid = lax.axis_index("c")
        sid = lax.axis_index("s")
        pltpu.sync_copy(zero_hbm, acc_sh.at[pl.ds(sid * ZROWS, ZROWS)])
        plsc.subcore_barrier()

        def body(i_vmem, m_vmem):
            pltpu.sync_copy(m_vmem, acc_sh.at[i_vmem.at[0]], add=True)

        pltpu.emit_pipeline(
            body,
            grid=(CL // GW,),
            in_specs=[
                pl.BlockSpec((1, GW), lambda i: (0, i)),
                pl.BlockSpec((GW, F), lambda i: (i, 0)),
            ],
            out_specs=[],
            core_axis_name=("c", "s"),
            dimension_semantics=(pltpu.PARALLEL,),
        )(dst_hbm, msg_hbm)

        plsc.subcore_barrier()
        pltpu.sync_copy(
            acc_sh.at[pl.ds(sid * ZROWS, ZROWS)],
            out_hbm.at[cid, pl.ds(sid * ZROWS, ZROWS)],
        )

    return _sc_scatter


def _add_body(*refs):
    o_ref = refs[-1]
    acc = jnp.sum(refs[0][...], axis=0)
    for r in refs[1:-1]:
        acc = acc + jnp.sum(r[...], axis=0)
    o_ref[...] = acc


def _tc_add(parts):
    return pl.pallas_call(
        _add_body,
        grid=(NPAD // ZROWS,),
        in_specs=[
            pl.BlockSpec((NC, ZROWS, F), lambda i: (0, i, 0)) for _ in parts
        ],
        out_specs=pl.BlockSpec((ZROWS, F), lambda i: (i, 0)),
        out_shape=jax.ShapeDtypeStruct((NPAD, F), jnp.float32),
    )(*parts)


def kernel(x, edge_index, edge_attr, W):
    src = edge_index[0].reshape(1, E)
    dst = edge_index[1].reshape(1, E)
    ea4 = jnp.pad(edge_attr, ((0, 0), (0, 1)))
    wall = (
        W.reshape(3, 3, 3, F, F)
        .transpose(1, 2, 3, 0, 4)
        .reshape(9 * F, 3 * F)
        .astype(jnp.bfloat16)
    )
    zero = jnp.zeros((ZROWS, F), jnp.float32)

    gather_k = _make_sc_gather()
    scatter_k = _make_sc_scatter()
    xgs = [
        gather_k(x, src[:, q * CL:(q + 1) * CL]) for q in range(NCHUNK)
    ]
    msgs = [
        _tc_gemm(xgs[q], ea4[q * CL:(q + 1) * CL], wall)
        for q in range(NCHUNK)
    ]
    parts = [
        scatter_k(msgs[q], dst[:, q * CL:(q + 1) * CL], zero)
        for q in range(NCHUNK)
    ]
    out = _tc_add(parts)
    return out[:N]


# speedup vs baseline: 1.0437x; 1.0038x over previous
import functools

import jax
import jax.numpy as jnp
from jax import lax
from jax.experimental import pallas as pl
from jax.experimental.pallas import tpu as pltpu
from jax.experimental.pallas import tpu_sc as plsc

N = 10000
E = 320000
F = 128
NPAD = 10240
NC = 2
NS = 16
CH = 80
PER_TILE_A = E // (NC * NS)
PER_TILE_C = E // (NC * NS)
ZROWS = NPAD // NS

GW = 128
NCHUNK = 2
CL = E // NCHUNK


@functools.cache
def _make_sc_gather():
    mesh = plsc.VectorSubcoreMesh(core_axis_name="c", subcore_axis_name="s")

    @functools.partial(
        pl.kernel,
        out_type=jax.ShapeDtypeStruct((CL, F), jnp.float32),
        mesh=mesh,
    )
    def _sc_gather(x_hbm, src_hbm, out_hbm):
        def body(i_vmem, o_vmem):
            pltpu.sync_copy(x_hbm.at[i_vmem.at[0]], o_vmem)

        pltpu.emit_pipeline(
            body,
            grid=(CL // GW,),
            in_specs=[pl.BlockSpec((1, GW), lambda i: (0, i))],
            out_specs=[pl.BlockSpec((GW, F), lambda i: (i, 0))],
            core_axis_name=("c", "s"),
            dimension_semantics=(pltpu.PARALLEL,),
        )(src_hbm, out_hbm)

    return _sc_gather


EB = 1280


NH = 2
HB = EB // NH


def _gemm_body(xg_ref, ea_ref, w_ref, msg_ref):
    for h in range(NH):
        rows = pl.ds(h * HB, HB)
        f = jnp.clip(ea_ref[rows, :], 0.0, 1.0)
        w0 = 0.5 * (1.0 - f) ** 2
        w1 = -f * f + f + 0.5
        w2 = 0.5 * f * f
        wb = (w0, w1, w2)
        xg = xg_ref[rows, :].astype(jnp.bfloat16)
        y = jnp.concatenate(
            [
                (wb[j][:, 1:2] * wb[l][:, 2:3]).astype(jnp.bfloat16) * xg
                for j in range(3)
                for l in range(3)
            ],
            axis=1,
        )
        g = lax.dot_general(
            y, w_ref[...], (((1,), (0,)), ((), ())),
            preferred_element_type=jnp.float32,
        )
        msg_ref[rows, :] = (
            w0[:, 0:1] * g[:, 0:128]
            + w1[:, 0:1] * g[:, 128:256]
            + w2[:, 0:1] * g[:, 256:384]
        )


def _tc_gemm(xg, ea4, wall):
    return pl.pallas_call(
        _gemm_body,
        grid=(CL // EB,),
        in_specs=[
            pl.BlockSpec((EB, F), lambda i: (i, 0)),
            pl.BlockSpec((EB, 4), lambda i: (i, 0)),
            pl.BlockSpec((9 * F, 3 * F), lambda i: (0, 0)),
        ],
        out_specs=pl.BlockSpec((EB, F), lambda i: (i, 0)),
        out_shape=jax.ShapeDtypeStruct((CL, F), jnp.float32),
    )(xg, ea4, wall)


@functools.cache
def _make_sc_scatter():
    mesh = plsc.VectorSubcoreMesh(core_axis_name="c", subcore_axis_name="s")

    @functools.partial(
        pl.kernel,
        out_type=jax.ShapeDtypeStruct((NC, NPAD, F), jnp.float32),
        mesh=mesh,
        scratch_types=[
            pltpu.VMEM_SHARED((NPAD, F), jnp.float32),
        ],
    )
    def _sc_scatter(msg_hbm, dst_hbm, zero_hbm, out_hbm, acc_sh):
        cid = lax.axis_index("c")
        sid = lax.axis_index("s")
        pltpu.sync_copy(zero_hbm, acc_sh.at[pl.ds(sid * ZROWS, ZROWS)])
        plsc.subcore_barrier()

        def body(i_vmem, m_vmem):
            pltpu.sync_copy(m_vmem, acc_sh.at[i_vmem.at[0]], add=True)

        pltpu.emit_pipeline(
            body,
            grid=(CL // GW,),
            in_specs=[
                pl.BlockSpec((1, GW), lambda i: (0, i)),
                pl.BlockSpec((GW, F), lambda i: (i, 0)),
            ],
            out_specs=[],
            core_axis_name=("c", "s"),
            dimension_semantics=(pltpu.PARALLEL,),
        )(dst_hbm, msg_hbm)

        plsc.subcore_barrier()
        pltpu.sync_copy(
            acc_sh.at[pl.ds(sid * ZROWS, ZROWS)],
            out_hbm.at[cid, pl.ds(sid * ZROWS, ZROWS)],
        )

    return _sc_scatter


def _add_body(*refs):
    o_ref = refs[-1]
    acc = jnp.sum(refs[0][...], axis=0)
    for r in refs[1:-1]:
        acc = acc + jnp.sum(r[...], axis=0)
    o_ref[...] = acc


def _tc_add(parts):
    return pl.pallas_call(
        _add_body,
        grid=(NPAD // ZROWS,),
        in_specs=[
            pl.BlockSpec((NC, ZROWS, F), lambda i: (0, i, 0)) for _ in parts
        ],
        out_specs=pl.BlockSpec((ZROWS, F), lambda i: (i, 0)),
        out_shape=jax.ShapeDtypeStruct((NPAD, F), jnp.float32),
    )(*parts)


def kernel(x, edge_index, edge_attr, W):
    src = edge_index[0].reshape(1, E)
    dst = edge_index[1].reshape(1, E)
    ea4 = jnp.pad(edge_attr, ((0, 0), (0, 1)))
    wall = (
        W.reshape(3, 3, 3, F, F)
        .transpose(1, 2, 3, 0, 4)
        .reshape(9 * F, 3 * F)
        .astype(jnp.bfloat16)
    )
    zero = jnp.zeros((ZROWS, F), jnp.float32)

    gather_k = _make_sc_gather()
    scatter_k = _make_sc_scatter()
    xgs = [
        gather_k(x, src[:, q * CL:(q + 1) * CL]) for q in range(NCHUNK)
    ]
    msgs = [
        _tc_gemm(xgs[q], ea4[q * CL:(q + 1) * CL], wall)
        for q in range(NCHUNK)
    ]
    parts = [
        scatter_k(msgs[q], dst[:, q * CL:(q + 1) * CL], zero)
        for q in range(NCHUNK)
    ]
    out = _tc_add(parts)
    return out[:N]
